# baseline (speedup 1.0000x reference)
import jax
import jax.numpy as jnp
from jax.experimental import pallas as pl
from jax.experimental.pallas import tpu as pltpu

B = 64
NB = 128
HID = 64
K = 8192
S = 64 * 64
BB = 8
R = BB * NB
NSTEPS = B // BB
BETA = 0.25


def _fused_kernel(x_ref, w1_ref, b1_ref, m2_ref, b2_ref, m3_ref, b3_ref,
                  m4_ref, b4_ref, c_ref, ct_ref, wdt_ref, bd_ref,
                  recon_ref, q_ref, idx_ref, loss_ref, rl_ref, vl_ref,
                  c2_ref, rl_acc, vl_acc, selt_ref, mnf_ref, mnl_ref):
    i = pl.program_id(0)

    @pl.when(i == 0)
    def _init():
        ct0 = ct_ref[...]
        c2_ref[...] = jnp.sum(ct0 * ct0, axis=0, keepdims=True)
        segr = jax.lax.broadcasted_iota(jnp.int32, (BB, R), 0)
        segc = jax.lax.broadcasted_iota(jnp.int32, (BB, R), 1)
        selt_ref[...] = (segc // NB == segr).astype(jnp.float32)
        lr = jax.lax.broadcasted_iota(jnp.int32, (1, R), 1)
        lmod = jax.lax.rem(lr, NB)
        mnf_ref[...] = (lmod != 0).astype(jnp.float32)
        mnl_ref[...] = (lmod != NB - 1).astype(jnp.float32)

    part = jnp.sum(x_ref[...].reshape(BB, S, NB), axis=1) * (1.0 / S)

    selt = selt_ref[...]
    mnf = mnf_ref[...]
    mnl = mnl_ref[...]

    tiled = jnp.concatenate([part] * BB, axis=1)
    h0 = jnp.sum(tiled * selt, axis=0, keepdims=True)

    def shift_prev(v):
        z = jnp.zeros((v.shape[0], 1), jnp.float32)
        return jnp.concatenate([z, v[:, :-1]], axis=1) * mnf

    def shift_next(v):
        z = jnp.zeros((v.shape[0], 1), jnp.float32)
        return jnp.concatenate([v[:, 1:], z], axis=1) * mnl

    w1 = w1_ref[...]
    h = (w1[:, 0:1] * shift_prev(h0)
         + w1[:, 1:2] * h0
         + w1[:, 2:3] * shift_next(h0)
         + b1_ref[...])
    h = jnp.maximum(h, 0.0)

    for m_ref, b_ref in ((m2_ref, b2_ref), (m3_ref, b3_ref),
                         (m4_ref, b4_ref)):
        h3 = jnp.concatenate(
            [shift_prev(h), h, shift_next(h)], axis=0)
        h = jnp.dot(m_ref[...], h3,
                    preferred_element_type=jnp.float32) + b_ref[...]
        h = jnp.maximum(h, 0.0)

    z = jax.lax.dot_general(
        selt, h, (((1,), (1,)), ((), ())),
        preferred_element_type=jnp.float32) * (1.0 / NB)

    zc = jnp.dot(z, ct_ref[...], preferred_element_type=jnp.float32)
    d = c2_ref[...] - 2.0 * zc
    dmin = jnp.min(d, axis=1, keepdims=True)
    lane = jax.lax.broadcasted_iota(jnp.int32, (BB, K), 1)
    idx = jnp.min(jnp.where(d == dmin, lane, K), axis=1,
                  keepdims=True)
    onehot = (lane == idx).astype(jnp.float32)
    q = jnp.dot(onehot, c_ref[...],
                preferred_element_type=jnp.float32)

    recon = jnp.dot(q, wdt_ref[...],
                    preferred_element_type=jnp.float32) + bd_ref[...]
    se = (recon - part) ** 2
    rl_part = jnp.sum(jnp.sum(se, axis=1, keepdims=True), axis=0,
                      keepdims=True)
    qe = (q - z) ** 2
    vl_part = jnp.sum(jnp.sum(qe, axis=1, keepdims=True), axis=0,
                      keepdims=True)

    recon_ref[...] = recon
    q_ref[...] = q
    idx_ref[...] = idx

    @pl.when(i == 0)
    def _first():
        rl_acc[...] = rl_part
        vl_acc[...] = vl_part

    @pl.when(i > 0)
    def _rest():
        rl_acc[...] += rl_part
        vl_acc[...] += vl_part

    @pl.when(i == NSTEPS - 1)
    def _emit():
        rl = rl_acc[...] * (1.0 / (B * NB))
        vl = vl_acc[...] * ((1.0 + BETA) / (B * HID))
        rl_ref[...] = rl
        vl_ref[...] = vl
        loss_ref[...] = rl + vl


def kernel(x, W1, b1, W2, b2, W3, b3, W4, b4, codebook, Wd, bd):
    xt = jnp.transpose(x, (0, 2, 3, 1)).reshape(B * S, NB)
    w1m = W1.reshape(HID, 3)
    m2 = jnp.transpose(W2, (2, 1, 0)).reshape(3 * HID, HID).T
    m3 = jnp.transpose(W3, (2, 1, 0)).reshape(3 * HID, HID).T
    m4 = jnp.transpose(W4, (2, 1, 0)).reshape(3 * HID, HID).T
    ct = codebook.T
    wdt = Wd.T
    b1c, b2c, b3c, b4c = (v.reshape(HID, 1) for v in (b1, b2, b3, b4))
    bdr = bd.reshape(1, NB)

    full = lambda shape: pl.BlockSpec(shape, lambda i: (0,) * len(shape))
    out_shapes = (
        jax.ShapeDtypeStruct((B, NB), jnp.float32),
        jax.ShapeDtypeStruct((B, HID), jnp.float32),
        jax.ShapeDtypeStruct((B, 1), jnp.int32),
        jax.ShapeDtypeStruct((1, 1), jnp.float32),
        jax.ShapeDtypeStruct((1, 1), jnp.float32),
        jax.ShapeDtypeStruct((1, 1), jnp.float32),
    )
    recon, q, idx, loss, rl, vl = pl.pallas_call(
        _fused_kernel,
        grid=(NSTEPS,),
        in_specs=[
            pl.BlockSpec((BB * S, NB), lambda i: (i, 0)),
            full((HID, 3)), full((HID, 1)),
            full((HID, 3 * HID)), full((HID, 1)),
            full((HID, 3 * HID)), full((HID, 1)),
            full((HID, 3 * HID)), full((HID, 1)),
            full((K, HID)), full((HID, K)),
            full((HID, NB)), full((1, NB)),
        ],
        out_specs=(
            pl.BlockSpec((BB, NB), lambda i: (i, 0)),
            pl.BlockSpec((BB, HID), lambda i: (i, 0)),
            pl.BlockSpec((BB, 1), lambda i: (i, 0)),
            full((1, 1)), full((1, 1)), full((1, 1)),
        ),
        out_shape=out_shapes,
        scratch_shapes=[pltpu.VMEM((1, K), jnp.float32),
                        pltpu.VMEM((1, 1), jnp.float32),
                        pltpu.VMEM((1, 1), jnp.float32),
                        pltpu.VMEM((BB, R), jnp.float32),
                        pltpu.VMEM((1, R), jnp.float32),
                        pltpu.VMEM((1, R), jnp.float32)],
    )(xt, w1m, b1c, m2, b2c, m3, b3c, m4, b4c, codebook, ct, wdt, bdr)

    return (recon, q[:, None, :], idx, loss[0, 0], rl[0, 0], vl[0, 0])


# speedup vs baseline: 1.0032x; 1.0032x over previous
import jax
import jax.numpy as jnp
from jax.experimental import pallas as pl
from jax.experimental.pallas import tpu as pltpu

B = 64
NB = 128
HID = 64
K = 8192
S = 64 * 64
BB = 8
R = BB * NB
NSTEPS = B // BB
BETA = 0.25


def _fused_kernel(x_ref, w1_ref, b1_ref, m2_ref, b2_ref, m3_ref, b3_ref,
                  m4_ref, b4_ref, c_ref, ct_ref, wdt_ref, bd_ref,
                  recon_ref, q_ref, idx_ref, loss_ref, rl_ref, vl_ref,
                  c2_ref, rl_acc, vl_acc, selt_ref, mnf_ref, mnl_ref):
    i = pl.program_id(0)

    @pl.when(i == 0)
    def _init():
        ct0 = ct_ref[...]
        c2_ref[...] = jnp.sum(ct0 * ct0, axis=0, keepdims=True)
        segr = jax.lax.broadcasted_iota(jnp.int32, (BB, R), 0)
        segc = jax.lax.broadcasted_iota(jnp.int32, (BB, R), 1)
        selt_ref[...] = (segc // NB == segr).astype(jnp.float32)
        lr = jax.lax.broadcasted_iota(jnp.int32, (1, R), 1)
        lmod = jax.lax.rem(lr, NB)
        mnf_ref[...] = (lmod != 0).astype(jnp.float32)
        mnl_ref[...] = (lmod != NB - 1).astype(jnp.float32)

    part = jnp.sum(x_ref[...].reshape(BB, S, NB), axis=1) * (1.0 / S)

    selt = selt_ref[...]
    mnf = mnf_ref[...]
    mnl = mnl_ref[...]

    tiled = jnp.concatenate([part] * BB, axis=1)
    h0 = jnp.sum(tiled * selt, axis=0, keepdims=True)

    def shift_prev(v):
        z = jnp.zeros((v.shape[0], 1), jnp.float32)
        return jnp.concatenate([z, v[:, :-1]], axis=1) * mnf

    def shift_next(v):
        z = jnp.zeros((v.shape[0], 1), jnp.float32)
        return jnp.concatenate([v[:, 1:], z], axis=1) * mnl

    w1 = w1_ref[...]
    h = (w1[:, 0:1] * shift_prev(h0)
         + w1[:, 1:2] * h0
         + w1[:, 2:3] * shift_next(h0)
         + b1_ref[...])
    h = jnp.maximum(h, 0.0)

    for m_ref, b_ref in ((m2_ref, b2_ref), (m3_ref, b3_ref),
                         (m4_ref, b4_ref)):
        h3 = jnp.concatenate(
            [shift_prev(h), h, shift_next(h)], axis=0)
        h = jnp.dot(m_ref[...], h3,
                    preferred_element_type=jnp.float32) + b_ref[...]
        h = jnp.maximum(h, 0.0)

    z = jax.lax.dot_general(
        selt, h, (((1,), (1,)), ((), ())),
        preferred_element_type=jnp.float32) * (1.0 / NB)

    zz = jnp.sum(z * z, axis=1, keepdims=True)
    zc = jnp.dot(z, ct_ref[...], preferred_element_type=jnp.float32)
    d = zz - 2.0 * zc + c2_ref[...]
    dmin = jnp.min(d, axis=1, keepdims=True)
    lane = jax.lax.broadcasted_iota(jnp.int32, (BB, K), 1)
    idx = jnp.min(jnp.where(d == dmin, lane, K), axis=1,
                  keepdims=True)
    onehot = (lane == idx).astype(jnp.float32)
    q = jnp.dot(onehot, c_ref[...],
                preferred_element_type=jnp.float32)

    recon = jnp.dot(q, wdt_ref[...],
                    preferred_element_type=jnp.float32) + bd_ref[...]
    se = (recon - part) ** 2
    rl_part = jnp.sum(jnp.sum(se, axis=1, keepdims=True), axis=0,
                      keepdims=True)
    qe = (q - z) ** 2
    vl_part = jnp.sum(jnp.sum(qe, axis=1, keepdims=True), axis=0,
                      keepdims=True)

    recon_ref[...] = recon
    q_ref[...] = q
    idx_ref[...] = idx

    @pl.when(i == 0)
    def _first():
        rl_acc[...] = rl_part
        vl_acc[...] = vl_part

    @pl.when(i > 0)
    def _rest():
        rl_acc[...] += rl_part
        vl_acc[...] += vl_part

    @pl.when(i == NSTEPS - 1)
    def _emit():
        rl = rl_acc[...] * (1.0 / (B * NB))
        vl = vl_acc[...] * ((1.0 + BETA) / (B * HID))
        rl_ref[...] = rl
        vl_ref[...] = vl
        loss_ref[...] = rl + vl


def kernel(x, W1, b1, W2, b2, W3, b3, W4, b4, codebook, Wd, bd):
    xt = jnp.transpose(x, (0, 2, 3, 1)).reshape(B * S, NB)
    w1m = W1.reshape(HID, 3)
    m2 = jnp.transpose(W2, (2, 1, 0)).reshape(3 * HID, HID).T
    m3 = jnp.transpose(W3, (2, 1, 0)).reshape(3 * HID, HID).T
    m4 = jnp.transpose(W4, (2, 1, 0)).reshape(3 * HID, HID).T
    ct = codebook.T
    wdt = Wd.T
    b1c, b2c, b3c, b4c = (v.reshape(HID, 1) for v in (b1, b2, b3, b4))
    bdr = bd.reshape(1, NB)

    full = lambda shape: pl.BlockSpec(shape, lambda i: (0,) * len(shape))
    out_shapes = (
        jax.ShapeDtypeStruct((B, NB), jnp.float32),
        jax.ShapeDtypeStruct((B, HID), jnp.float32),
        jax.ShapeDtypeStruct((B, 1), jnp.int32),
        jax.ShapeDtypeStruct((1, 1), jnp.float32),
        jax.ShapeDtypeStruct((1, 1), jnp.float32),
        jax.ShapeDtypeStruct((1, 1), jnp.float32),
    )
    recon, q, idx, loss, rl, vl = pl.pallas_call(
        _fused_kernel,
        grid=(NSTEPS,),
        in_specs=[
            pl.BlockSpec((BB * S, NB), lambda i: (i, 0)),
            full((HID, 3)), full((HID, 1)),
            full((HID, 3 * HID)), full((HID, 1)),
            full((HID, 3 * HID)), full((HID, 1)),
            full((HID, 3 * HID)), full((HID, 1)),
            full((K, HID)), full((HID, K)),
            full((HID, NB)), full((1, NB)),
        ],
        out_specs=(
            pl.BlockSpec((BB, NB), lambda i: (i, 0)),
            pl.BlockSpec((BB, HID), lambda i: (i, 0)),
            pl.BlockSpec((BB, 1), lambda i: (i, 0)),
            full((1, 1)), full((1, 1)), full((1, 1)),
        ),
        out_shape=out_shapes,
        scratch_shapes=[pltpu.VMEM((1, K), jnp.float32),
                        pltpu.VMEM((1, 1), jnp.float32),
                        pltpu.VMEM((1, 1), jnp.float32),
                        pltpu.VMEM((BB, R), jnp.float32),
                        pltpu.VMEM((1, R), jnp.float32),
                        pltpu.VMEM((1, R), jnp.float32)],
    )(xt, w1m, b1c, m2, b2c, m3, b3c, m4, b4c, codebook, ct, wdt, bdr)

    return (recon, q[:, None, :], idx, loss[0, 0], rl[0, 0], vl[0, 0])
